# baseline (speedup 1.0000x reference)
import jax
import jax.numpy as jnp
from jax import lax
from jax.experimental import pallas as pl
from jax.experimental.pallas import tpu as pltpu


def _tdo_body(lam_ref, wq_ref, wk_ref, qt_ref, qp_ref, kt_ref, kp_ref,
              v_ref, o_ref):
    hd = qt_ref.shape[-2]
    nf = wq_ref.shape[0]

    def lambda_row(x, xp, w):
        scores = (
            lax.dot_general(w[:, :hd], x, (((1,), (0,)), ((), ())),
                            preferred_element_type=jnp.float32)
            + lax.dot_general(w[:, hd:], xp, (((1,), (0,)), ((), ())),
                              preferred_element_type=jnp.float32))
        m = jnp.max(scores, axis=0, keepdims=True)
        iota = lax.broadcasted_iota(jnp.int32, scores.shape, 0)
        idx = jnp.min(jnp.where(scores == m, iota, nf), axis=0,
                      keepdims=True)
        lam = jnp.zeros((1, idx.shape[1]), jnp.float32)
        for n in range(nf):
            lam = jnp.where(idx == n, lam_ref[0, n], lam)
        return lam

    for j in range(qt_ref.shape[1]):
        qt = qt_ref[0, j]
        qp = qp_ref[0, j]
        kt = kt_ref[0, j]
        kp = kp_ref[0, j]
        v = v_ref[0, j]
        q_diff = qt - lambda_row(qt, qp, wq_ref[...]) * qp
        k_diff = kt - lambda_row(kt, kp, wk_ref[...]) * kp
        kvt = lax.dot_general(v, k_diff, (((1,), (1,)), ((), ())),
                              preferred_element_type=jnp.float32)
        o_ref[0, j] = jnp.dot(kvt, q_diff,
                              preferred_element_type=jnp.float32)


def kernel(Q_t, Q_prime_t, K_t, K_prime_t, V, lambdas, Wq, Wk):
    b, h, s, hd = Q_t.shape
    nf = lambdas.shape[0]
    dim = Wq.shape[1]

    qt = jnp.swapaxes(Q_t, 2, 3)
    qp = jnp.swapaxes(Q_prime_t, 2, 3)
    kt = jnp.swapaxes(K_t, 2, 3)
    kp = jnp.swapaxes(K_prime_t, 2, 3)
    v = jnp.swapaxes(V, 2, 3)
    lam = lambdas.reshape(1, nf)

    hb = 4
    hg = h // hb
    big = pl.BlockSpec((1, hb, hd, s), lambda i: (i // hg, i % hg, 0, 0))
    lam_spec = pl.BlockSpec(memory_space=pltpu.SMEM)
    w_spec = pl.BlockSpec((nf, dim), lambda i: (0, 0))

    out = pl.pallas_call(
        _tdo_body,
        out_shape=jax.ShapeDtypeStruct((b, h, hd, s), jnp.float32),
        grid=(b * hg,),
        in_specs=[lam_spec, w_spec, w_spec, big, big, big, big, big],
        out_specs=big,
        compiler_params=pltpu.CompilerParams(
            dimension_semantics=("parallel",),
        ),
        name="token_diff_op",
    )(lam, Wq, Wk, qt, qp, kt, kp, v)
    return jnp.swapaxes(out, 2, 3)


# speedup vs baseline: 1.0045x; 1.0045x over previous
import jax
import jax.numpy as jnp
from jax import lax
from jax.experimental import pallas as pl
from jax.experimental.pallas import tpu as pltpu


def _tdo_body(lam_ref, wq_ref, wk_ref, qt_ref, qp_ref, kt_ref, kp_ref,
              v_ref, o_ref):
    hd = qt_ref.shape[-2]
    nf = wq_ref.shape[0]

    def lambda_row(x, xp, w):
        scores = (
            lax.dot_general(w[:, :hd], x, (((1,), (0,)), ((), ())),
                            preferred_element_type=jnp.float32)
            + lax.dot_general(w[:, hd:], xp, (((1,), (0,)), ((), ())),
                              preferred_element_type=jnp.float32))
        m = jnp.max(scores, axis=0, keepdims=True)
        iota = lax.broadcasted_iota(jnp.int32, scores.shape, 0)
        idx = jnp.min(jnp.where(scores == m, iota, nf), axis=0,
                      keepdims=True)
        lam = jnp.zeros((1, idx.shape[1]), jnp.float32)
        for n in range(nf):
            lam = jnp.where(idx == n, lam_ref[0, n], lam)
        return lam

    for j in range(qt_ref.shape[1]):
        qt = qt_ref[0, j]
        qp = qp_ref[0, j]
        kt = kt_ref[0, j]
        kp = kp_ref[0, j]
        v = v_ref[0, j]
        q_diff = qt - lambda_row(qt, qp, wq_ref[...]) * qp
        k_diff = kt - lambda_row(kt, kp, wk_ref[...]) * kp
        kvt = lax.dot_general(v, k_diff, (((1,), (1,)), ((), ())),
                              preferred_element_type=jnp.float32)
        o_ref[0, j] = jnp.dot(kvt, q_diff,
                              preferred_element_type=jnp.float32)


def kernel(Q_t, Q_prime_t, K_t, K_prime_t, V, lambdas, Wq, Wk):
    b, h, s, hd = Q_t.shape
    nf = lambdas.shape[0]
    dim = Wq.shape[1]

    qt = jnp.swapaxes(Q_t, 2, 3)
    qp = jnp.swapaxes(Q_prime_t, 2, 3)
    kt = jnp.swapaxes(K_t, 2, 3)
    kp = jnp.swapaxes(K_prime_t, 2, 3)
    v = jnp.swapaxes(V, 2, 3)
    lam = lambdas.reshape(1, nf)

    hb = 2
    hg = h // hb
    big = pl.BlockSpec((1, hb, hd, s), lambda i: (i // hg, i % hg, 0, 0))
    lam_spec = pl.BlockSpec(memory_space=pltpu.SMEM)
    w_spec = pl.BlockSpec((nf, dim), lambda i: (0, 0))

    out = pl.pallas_call(
        _tdo_body,
        out_shape=jax.ShapeDtypeStruct((b, h, hd, s), jnp.float32),
        grid=(b * hg,),
        in_specs=[lam_spec, w_spec, w_spec, big, big, big, big, big],
        out_specs=big,
        compiler_params=pltpu.CompilerParams(
            dimension_semantics=("parallel",),
        ),
        name="token_diff_op",
    )(lam, Wq, Wk, qt, qp, kt, kp, v)
    return jnp.swapaxes(out, 2, 3)


# speedup vs baseline: 1.0102x; 1.0057x over previous
import jax
import jax.numpy as jnp
from jax import lax
from jax.experimental import pallas as pl
from jax.experimental.pallas import tpu as pltpu


def _tdo_body(lam_ref, wq_ref, wk_ref, qt_ref, qp_ref, kt_ref, kp_ref,
              v_ref, o_ref):
    hd = qt_ref.shape[-2]
    nf = wq_ref.shape[0]

    def lambda_row(x, xp, w):
        scores = (
            lax.dot_general(w[:, :hd], x, (((1,), (0,)), ((), ())),
                            preferred_element_type=jnp.float32)
            + lax.dot_general(w[:, hd:], xp, (((1,), (0,)), ((), ())),
                              preferred_element_type=jnp.float32))
        m = jnp.max(scores, axis=0, keepdims=True)
        iota = lax.broadcasted_iota(jnp.int32, scores.shape, 0)
        idx = jnp.min(jnp.where(scores == m, iota, nf), axis=0,
                      keepdims=True)
        lam = jnp.zeros((1, idx.shape[1]), jnp.float32)
        for n in range(nf):
            lam = jnp.where(idx == n, lam_ref[0, n], lam)
        return lam

    for j in range(qt_ref.shape[1]):
        qt = qt_ref[0, j]
        qp = qp_ref[0, j]
        kt = kt_ref[0, j]
        kp = kp_ref[0, j]
        v = v_ref[0, j]
        q_diff = qt - lambda_row(qt, qp, wq_ref[...]) * qp
        k_diff = kt - lambda_row(kt, kp, wk_ref[...]) * kp
        kvt = lax.dot_general(v, k_diff, (((1,), (1,)), ((), ())),
                              preferred_element_type=jnp.float32)
        o_ref[0, j] = jnp.dot(kvt, q_diff,
                              preferred_element_type=jnp.float32)


def kernel(Q_t, Q_prime_t, K_t, K_prime_t, V, lambdas, Wq, Wk):
    b, h, s, hd = Q_t.shape
    nf = lambdas.shape[0]
    dim = Wq.shape[1]

    qt = jnp.swapaxes(Q_t, 2, 3)
    qp = jnp.swapaxes(Q_prime_t, 2, 3)
    kt = jnp.swapaxes(K_t, 2, 3)
    kp = jnp.swapaxes(K_prime_t, 2, 3)
    v = jnp.swapaxes(V, 2, 3)
    lam = lambdas.reshape(1, nf)

    hb = 2 if h % 2 == 0 else 1
    hg = h // hb
    big = pl.BlockSpec((1, hb, hd, s), lambda i: (i // hg, i % hg, 0, 0))
    lam_spec = pl.BlockSpec(memory_space=pltpu.SMEM)
    w_spec = pl.BlockSpec((nf, dim), lambda i: (0, 0))

    out = pl.pallas_call(
        _tdo_body,
        out_shape=jax.ShapeDtypeStruct((b, h, hd, s), jnp.float32),
        grid=(b * hg,),
        in_specs=[lam_spec, w_spec, w_spec, big, big, big, big, big],
        out_specs=big,
        compiler_params=pltpu.CompilerParams(
            dimension_semantics=("parallel",),
        ),
        name="token_diff_op",
    )(lam, Wq, Wk, qt, qp, kt, kp, v)
    return jnp.swapaxes(out, 2, 3)
